# Initial kernel scaffold; baseline (speedup 1.0000x reference)
import functools

import jax
import jax.numpy as jnp
from jax import lax
from jax.experimental import pallas as pl
from jax.experimental.pallas import tpu as pltpu
from jax.experimental.pallas import tpu_sc as plsc

N = 10000
E = 320000
D = 128

NC = 2
NS = 16
NW = NC * NS
EPW = E // NW
C = 80
NCHUNK = EPW // C
ROWS_PER_TILE = N // NS
ZR = 125


def _spmm_sc(src, dst, w, x):
    mesh = plsc.VectorSubcoreMesh(core_axis_name="c", subcore_axis_name="s")

    @functools.partial(
        pl.kernel,
        out_type=jax.ShapeDtypeStruct((NC, N, D), jnp.float32),
        mesh=mesh,
        scratch_types=[
            pltpu.VMEM((C,), jnp.int32),
            pltpu.VMEM((C,), jnp.int32),
            pltpu.VMEM((C,), jnp.float32),
            pltpu.VMEM((C, D), jnp.float32),
            pltpu.VMEM((ZR, D), jnp.float32),
            pltpu.VMEM_SHARED((N, D), jnp.float32),
            pltpu.SemaphoreType.DMA,
        ],
    )
    def spmm(src_hbm, dst_hbm, w_hbm, x_hbm, out_hbm,
             src_v, dst_v, w_v, rows_v, zbuf, acc, sem):
        cid = lax.axis_index("c")
        sid = lax.axis_index("s")
        wid = sid * NC + cid

        zvec = jnp.zeros((16,), jnp.float32)

        def zero_row(r, _):
            for j in range(D // 16):
                zbuf[r, pl.ds(j * 16, 16)] = zvec
            return 0

        lax.fori_loop(0, ZR, zero_row, 0)
        for k in range(ROWS_PER_TILE // ZR):
            r0 = sid * ROWS_PER_TILE + k * ZR
            pltpu.sync_copy(zbuf, acc.at[pl.ds(r0, ZR)])
        plsc.subcore_barrier()

        def chunk_body(i, _):
            base = wid * EPW + i * C
            pltpu.sync_copy(src_hbm.at[pl.ds(base, C)], src_v)
            pltpu.sync_copy(dst_hbm.at[pl.ds(base, C)], dst_v)
            pltpu.sync_copy(w_hbm.at[pl.ds(base, C)], w_v)
            pltpu.async_copy(x_hbm.at[src_v], rows_v, sem).wait()
            for k in range(C):
                wb = plsc.load_gather(
                    w_v, [jnp.full((16,), k, jnp.int32)])
                for j in range(D // 16):
                    sl = pl.ds(j * 16, 16)
                    rows_v[k, sl] = rows_v[k, sl] * wb
            pltpu.sync_copy(rows_v, acc.at[dst_v], add=True)
            return 0

        lax.fori_loop(0, NCHUNK, chunk_body, 0)
        plsc.subcore_barrier()

        for k in range(ROWS_PER_TILE // ZR):
            r0 = sid * ROWS_PER_TILE + k * ZR
            pltpu.sync_copy(acc.at[pl.ds(r0, ZR)],
                            out_hbm.at[cid, pl.ds(r0, ZR)])

    return spmm(src, dst, w, x)


BLK = 1000


def _linear_tc(partials, W, b2d):
    def body(p_ref, w_ref, b_ref, o_ref):
        s = p_ref[0] + p_ref[1]
        o_ref[...] = jnp.dot(
            s, w_ref[...], preferred_element_type=jnp.float32) + b_ref[...]

    return pl.pallas_call(
        body,
        grid=(N // BLK,),
        in_specs=[
            pl.BlockSpec((NC, BLK, D), lambda i: (0, i, 0)),
            pl.BlockSpec((D, D), lambda i: (0, 0)),
            pl.BlockSpec((1, D), lambda i: (0, 0)),
        ],
        out_specs=pl.BlockSpec((BLK, D), lambda i: (i, 0)),
        out_shape=jax.ShapeDtypeStruct((N, D), jnp.float32),
    )(partials, W, b2d)


@jax.jit
def kernel(x, edge_index, edge_weight, W, b):
    dst = edge_index[0]
    src = edge_index[1]
    partials = _spmm_sc(src, dst, edge_weight, x)
    return _linear_tc(partials, W, b.reshape(1, D))


# speedup vs baseline: 3.0877x; 3.0877x over previous
import functools

import jax
import jax.numpy as jnp
from jax import lax
from jax.experimental import pallas as pl
from jax.experimental.pallas import tpu as pltpu
from jax.experimental.pallas import tpu_sc as plsc

N = 10000
E = 320000
D = 128

NC = 2
NS = 16
NW = NC * NS
EPW = E // NW
C = 80
NCHUNK = EPW // C
NP = 10240
ROWS_PER_TILE = NP // NS
ZR = 128


def _spmm_sc(src, dst, wexp, x):
    mesh = plsc.VectorSubcoreMesh(core_axis_name="c", subcore_axis_name="s")

    @functools.partial(
        pl.kernel,
        out_type=jax.ShapeDtypeStruct((NC, NP, D), jnp.float32),
        mesh=mesh,
        scratch_types=[
            pltpu.VMEM((C,), jnp.int32),
            pltpu.VMEM((C,), jnp.int32),
            pltpu.VMEM((C, 16), jnp.float32),
            pltpu.VMEM((C, D), jnp.float32),
            pltpu.VMEM((ZR, D), jnp.float32),
            pltpu.VMEM_SHARED((NP, D), jnp.float32),
            pltpu.SemaphoreType.DMA,
        ],
    )
    def spmm(src_hbm, dst_hbm, w_hbm, x_hbm, out_hbm,
             src_v, dst_v, w_v, rows_v, zbuf, acc, sem):
        cid = lax.axis_index("c")
        sid = lax.axis_index("s")
        wid = sid * NC + cid

        zvec = jnp.zeros((16,), jnp.float32)

        def zero_row(r, _):
            for j in range(D // 16):
                zbuf[r, pl.ds(j * 16, 16)] = zvec
            return 0

        lax.fori_loop(0, ZR, zero_row, 0)
        for k in range(ROWS_PER_TILE // ZR):
            r0 = sid * ROWS_PER_TILE + k * ZR
            pltpu.sync_copy(zbuf, acc.at[pl.ds(r0, ZR)])
        plsc.subcore_barrier()

        def chunk_body(i, _):
            base = wid * EPW + i * C
            pltpu.sync_copy(src_hbm.at[pl.ds(base, C)], src_v)
            pltpu.sync_copy(dst_hbm.at[pl.ds(base, C)], dst_v)
            pltpu.sync_copy(w_hbm.at[pl.ds(base, C)], w_v)
            pltpu.async_copy(x_hbm.at[src_v], rows_v, sem).wait()
            for k in range(C):
                wb = w_v[k]
                for j in range(D // 16):
                    sl = pl.ds(j * 16, 16)
                    rows_v[k, sl] = rows_v[k, sl] * wb
            pltpu.sync_copy(rows_v, acc.at[dst_v], add=True)
            return 0

        lax.fori_loop(0, NCHUNK, chunk_body, 0)
        plsc.subcore_barrier()

        for k in range(ROWS_PER_TILE // ZR):
            r0 = sid * ROWS_PER_TILE + k * ZR
            pltpu.sync_copy(acc.at[pl.ds(r0, ZR)],
                            out_hbm.at[cid, pl.ds(r0, ZR)])

    return spmm(src, dst, wexp, x)


BLK = 1000


def _linear_tc(partials, W, b2d):
    def body(p_ref, w_ref, b_ref, o_ref):
        s = p_ref[0] + p_ref[1]
        o_ref[...] = jnp.dot(
            s, w_ref[...], preferred_element_type=jnp.float32) + b_ref[...]

    return pl.pallas_call(
        body,
        grid=(N // BLK,),
        in_specs=[
            pl.BlockSpec((NC, BLK, D), lambda i: (0, i, 0)),
            pl.BlockSpec((D, D), lambda i: (0, 0)),
            pl.BlockSpec((1, D), lambda i: (0, 0)),
        ],
        out_specs=pl.BlockSpec((BLK, D), lambda i: (i, 0)),
        out_shape=jax.ShapeDtypeStruct((N, D), jnp.float32),
    )(partials, W, b2d)


@jax.jit
def kernel(x, edge_index, edge_weight, W, b):
    dst = edge_index[0]
    src = edge_index[1]
    wexp = jnp.broadcast_to(edge_weight[:, None], (E, 16))
    partials = _spmm_sc(src, dst, wexp, x)
    return _linear_tc(partials, W, b.reshape(1, D))


# speedup vs baseline: 3.2164x; 1.0417x over previous
import functools

import jax
import jax.numpy as jnp
from jax import lax
from jax.experimental import pallas as pl
from jax.experimental.pallas import tpu as pltpu
from jax.experimental.pallas import tpu_sc as plsc

N = 10000
E = 320000
D = 128
DH = D // 2

NC = 2
NS = 16
CB = 128
CH = 160
PADE = NS * CH * CB
NP = 10240
TRASH = 10200
ROWS_PER_TILE = NP // NS
ZR = 128
NBUF = 4
NGRP = CH // NBUF
PD = 2


def _spmm_sc(srcp, dstp, wexpp, xflat):
    mesh = plsc.VectorSubcoreMesh(core_axis_name="c", subcore_axis_name="s")

    @functools.partial(
        pl.kernel,
        out_type=jax.ShapeDtypeStruct((NC, NP, DH), jnp.float32),
        mesh=mesh,
        scratch_types=[
            pltpu.VMEM((CH, CB), jnp.int32),
            pltpu.VMEM((CH, CB), jnp.int32),
            [pltpu.VMEM((CB, 16), jnp.float32) for _ in range(NBUF)],
            [pltpu.VMEM((CB, DH), jnp.float32) for _ in range(NBUF)],
            pltpu.VMEM_SHARED((NP, DH), jnp.float32),
            [pltpu.SemaphoreType.DMA for _ in range(NBUF)],
            [pltpu.SemaphoreType.DMA for _ in range(NBUF)],
            [pltpu.SemaphoreType.DMA for _ in range(NBUF)],
        ],
        compiler_params=pltpu.CompilerParams(use_tc_tiling_on_sc=False),
    )
    def spmm(src_hbm, dst_hbm, wexp_hbm, x_hbm, out_hbm,
             src_slab, dst_slab, wexp_v, rows_v, acc,
             gsem, ssem, wsem):
        cid = lax.axis_index("c")
        sid = lax.axis_index("s")

        pltpu.sync_copy(src_hbm.at[cid, sid], src_slab)
        pltpu.sync_copy(dst_hbm.at[sid], dst_slab)

        zvec = jnp.zeros((16,), jnp.float32)

        def zero_row(r, _):
            for j in range(DH // 16):
                rows_v[0][r, pl.ds(j * 16, 16)] = zvec
            return 0

        lax.fori_loop(0, CB, zero_row, 0)
        for k in range(ROWS_PER_TILE // ZR):
            r0 = sid * ROWS_PER_TILE + k * ZR
            pltpu.sync_copy(rows_v[0], acc.at[pl.ds(r0, ZR)])
        plsc.subcore_barrier()

        def start_fetch(i, b):
            pltpu.async_copy(x_hbm.at[src_slab.at[i]], rows_v[b], gsem[b])
            pltpu.async_copy(
                wexp_hbm.at[pl.ds((sid * CH + i) * CB, CB)],
                wexp_v[b], wsem[b])

        def wait_fetch(i, b):
            pltpu.make_async_copy(
                x_hbm.at[src_slab.at[i]], rows_v[b], gsem[b]).wait()
            pltpu.make_async_copy(
                wexp_hbm.at[pl.ds((sid * CH + i) * CB, CB)],
                wexp_v[b], wsem[b]).wait()

        def start_scatter(i, b):
            pltpu.async_copy(
                rows_v[b], acc.at[dst_slab.at[i]], ssem[b], add=True)

        def wait_scatter(i, b):
            pltpu.make_async_copy(
                rows_v[b], acc.at[dst_slab.at[i]], ssem[b]).wait()

        for b in range(PD):
            start_fetch(b, b)

        def group_body(g, _):
            for b in range(NBUF):
                i = g * NBUF + b
                wait_fetch(i, b)

                def scale_body(t, _, b=b):
                    for kk in range(8):
                        k = t * 8 + kk
                        wb = wexp_v[b][k]
                        for j in range(DH // 16):
                            sl = pl.ds(j * 16, 16)
                            rows_v[b][k, sl] = rows_v[b][k, sl] * wb
                    return 0

                lax.fori_loop(0, CB // 8, scale_body, 0)
                start_scatter(i, b)

                j_next = i + PD
                bj = (b + PD) % NBUF

                @pl.when(j_next < CH)
                def _():
                    @pl.when(j_next >= NBUF)
                    def _():
                        wait_scatter(j_next - NBUF, bj)
                    start_fetch(j_next, bj)
            return 0

        lax.fori_loop(0, NGRP, group_body, 0)
        for k in range(CH - NBUF, CH):
            wait_scatter(k, k % NBUF)
        plsc.subcore_barrier()

        for k in range(ROWS_PER_TILE // ZR):
            r0 = sid * ROWS_PER_TILE + k * ZR
            pltpu.sync_copy(acc.at[pl.ds(r0, ZR)],
                            out_hbm.at[cid, pl.ds(r0, ZR)])

    return spmm(srcp, dstp, wexpp, xflat)


BLK = 1000


def _linear_tc(halves, W, b2d):
    def body(p_ref, w_ref, b_ref, o_ref):
        s = jnp.concatenate([p_ref[0], p_ref[1]], axis=1)
        o_ref[...] = jnp.dot(
            s, w_ref[...], preferred_element_type=jnp.float32) + b_ref[...]

    return pl.pallas_call(
        body,
        grid=(N // BLK,),
        in_specs=[
            pl.BlockSpec((NC, BLK, DH), lambda i: (0, i, 0)),
            pl.BlockSpec((D, D), lambda i: (0, 0)),
            pl.BlockSpec((1, D), lambda i: (0, 0)),
        ],
        out_specs=pl.BlockSpec((BLK, D), lambda i: (i, 0)),
        out_shape=jax.ShapeDtypeStruct((N, D), jnp.float32),
    )(halves, W, b2d)


@jax.jit
def kernel(x, edge_index, edge_weight, W, b):
    dst = edge_index[0]
    src = edge_index[1]
    npad = PADE - E
    src_pad = jnp.concatenate([src, jnp.zeros((npad,), jnp.int32)])
    srcp = jnp.stack([src_pad, src_pad + N]).reshape(NC, NS, CH, CB)
    dstp = jnp.concatenate(
        [dst, jnp.full((npad,), TRASH, jnp.int32)]).reshape(NS, CH, CB)
    wpad = jnp.concatenate([edge_weight, jnp.zeros((npad,), jnp.float32)])
    wexpp = jnp.broadcast_to(wpad[:, None], (PADE, 16))
    xflat = jnp.concatenate([x[:, :DH], x[:, DH:]], axis=0)
    halves = _spmm_sc(srcp, dstp, wexpp, xflat)
    return _linear_tc(halves, W, b.reshape(1, D))


# speedup vs baseline: 4.3522x; 1.3531x over previous
import functools

import jax
import jax.numpy as jnp
from jax import lax
from jax.experimental import pallas as pl
from jax.experimental.pallas import tpu as pltpu
from jax.experimental.pallas import tpu_sc as plsc

N = 10000
E = 320000
D = 128
DH = D // 2

NC = 2
NS = 16
CB = 80
CH = 256
PADE = NS * CH * CB
NP = 10240
ROWS_PER_TILE = NP // NS
ZR = 128
NBUF = 4
NGRP = CH // NBUF
PD = 2


def _spmm_sc(packed, wexpp, xp):
    mesh = plsc.VectorSubcoreMesh(core_axis_name="c", subcore_axis_name="s")

    @functools.partial(
        pl.kernel,
        out_type=jax.ShapeDtypeStruct((NC, NP, DH), jnp.float32),
        mesh=mesh,
        scratch_types=[
            pltpu.VMEM((CH, CB), jnp.int32),
            [pltpu.VMEM((CB,), jnp.int32) for _ in range(NBUF)],
            [pltpu.VMEM((CB,), jnp.int32) for _ in range(NBUF)],
            [pltpu.VMEM((CB, 16), jnp.float32) for _ in range(2)],
            [pltpu.VMEM((CB, DH), jnp.float32) for _ in range(NBUF)],
            pltpu.VMEM_SHARED((NP, DH), jnp.float32),
            pltpu.VMEM_SHARED((NP, DH), jnp.float32),
            [pltpu.SemaphoreType.DMA for _ in range(NBUF)],
            [pltpu.SemaphoreType.DMA for _ in range(NBUF)],
            [pltpu.SemaphoreType.DMA for _ in range(2)],
        ],
        compiler_params=pltpu.CompilerParams(use_tc_tiling_on_sc=False),
    )
    def spmm(packed_hbm, wexp_hbm, xp_hbm, out_hbm,
             slab, src_v, dst_v, wexp_v, rows_v, xsh, acc,
             gsem, ssem, wsem):
        cid = lax.axis_index("c")
        sid = lax.axis_index("s")

        r0s = sid * ROWS_PER_TILE
        pltpu.sync_copy(xp_hbm.at[cid, pl.ds(r0s, ROWS_PER_TILE)],
                        xsh.at[pl.ds(r0s, ROWS_PER_TILE)])
        pltpu.sync_copy(packed_hbm.at[sid], slab)

        zvec = jnp.zeros((16,), jnp.float32)

        def zero_row(r, _):
            for j in range(DH // 16):
                rows_v[0][r, pl.ds(j * 16, 16)] = zvec
            return 0

        lax.fori_loop(0, CB, zero_row, 0)
        for k in range(ROWS_PER_TILE // CB):
            pltpu.sync_copy(rows_v[0], acc.at[pl.ds(r0s + k * CB, CB)])
        plsc.subcore_barrier()

        NV = CB // 16

        def unpack_src(i, bj):
            for g in range(NV):
                v = slab[i, pl.ds(g * 16, 16)]
                src_v[bj][pl.ds(g * 16, 16)] = v & 0xFFFF

        def unpack_dst(i, b):
            for g in range(NV):
                v = slab[i, pl.ds(g * 16, 16)]
                dst_v[b][pl.ds(g * 16, 16)] = lax.shift_right_logical(v, 16)

        def start_fetch(i, b):
            pltpu.async_copy(xsh.at[src_v[b]], rows_v[b], gsem[b])
            pltpu.async_copy(
                wexp_hbm.at[pl.ds((sid * CH + i) * CB, CB)],
                wexp_v[b % 2], wsem[b % 2])

        def wait_fetch(i, b):
            pltpu.make_async_copy(
                xsh.at[src_v[b]], rows_v[b], gsem[b]).wait()
            pltpu.make_async_copy(
                wexp_hbm.at[pl.ds((sid * CH + i) * CB, CB)],
                wexp_v[b % 2], wsem[b % 2]).wait()

        def start_scatter(i, b):
            pltpu.async_copy(
                rows_v[b], acc.at[dst_v[b]], ssem[b], add=True)

        def wait_scatter(i, b):
            pltpu.make_async_copy(
                rows_v[b], acc.at[dst_v[b]], ssem[b]).wait()

        for b in range(PD):
            unpack_src(b, b)
            start_fetch(b, b)

        def group_body(g, _):
            for b in range(NBUF):
                i = g * NBUF + b
                wait_fetch(i, b)

                def scale_body(t, _, b=b):
                    for kk in range(8):
                        k = t * 8 + kk
                        wb = wexp_v[b % 2][k]
                        for j in range(DH // 16):
                            sl = pl.ds(j * 16, 16)
                            rows_v[b][k, sl] = rows_v[b][k, sl] * wb
                    return 0

                lax.fori_loop(0, CB // 8, scale_body, 0)
                unpack_dst(i, b)
                start_scatter(i, b)

                j_next = i + PD
                bj = (b + PD) % NBUF

                @pl.when(j_next < CH)
                def _():
                    @pl.when(j_next >= NBUF)
                    def _():
                        wait_scatter(j_next - NBUF, bj)
                    unpack_src(j_next, bj)
                    start_fetch(j_next, bj)
            return 0

        lax.fori_loop(0, NGRP, group_body, 0)
        for k in range(CH - NBUF, CH):
            wait_scatter(k, k % NBUF)
        plsc.subcore_barrier()

        for k in range(ROWS_PER_TILE // ZR):
            r0 = r0s + k * ZR
            pltpu.sync_copy(acc.at[pl.ds(r0, ZR)],
                            out_hbm.at[cid, pl.ds(r0, ZR)])

    return spmm(packed, wexpp, xp)


BLK = 1000


def _linear_tc(halves, W, b2d):
    def body(p_ref, w_ref, b_ref, o_ref):
        s = jnp.concatenate([p_ref[0], p_ref[1]], axis=1)
        o_ref[...] = jnp.dot(
            s, w_ref[...], preferred_element_type=jnp.float32) + b_ref[...]

    return pl.pallas_call(
        body,
        grid=(N // BLK,),
        in_specs=[
            pl.BlockSpec((NC, BLK, DH), lambda i: (0, i, 0)),
            pl.BlockSpec((D, D), lambda i: (0, 0)),
            pl.BlockSpec((1, D), lambda i: (0, 0)),
        ],
        out_specs=pl.BlockSpec((BLK, D), lambda i: (i, 0)),
        out_shape=jax.ShapeDtypeStruct((N, D), jnp.float32),
    )(halves, W, b2d)


@jax.jit
def kernel(x, edge_index, edge_weight, W, b):
    dst = edge_index[0]
    src = edge_index[1]
    npad = PADE - E
    parange = jnp.arange(npad, dtype=jnp.int32)
    src_pad = jnp.concatenate([src, parange % N])
    dst_pad = jnp.concatenate([dst, N + parange % (NP - N)])
    packed = (
        jnp.left_shift(dst_pad, 16) | src_pad).reshape(NS, CH, CB)
    wpad = jnp.concatenate([edge_weight, jnp.zeros((npad,), jnp.float32)])
    wexpp = jnp.broadcast_to(wpad[:, None], (PADE, 16))
    xp = jnp.concatenate([
        jnp.stack([x[:, :DH], x[:, DH:]]),
        jnp.zeros((NC, NP - N, DH), jnp.float32)], axis=1)
    halves = _spmm_sc(packed, wexpp, xp)
    return _linear_tc(halves, W, b.reshape(1, D))
